# Initial kernel scaffold; baseline (speedup 1.0000x reference)
import jax
import jax.numpy as jnp
from jax.experimental import pallas as pl


def kernel(rna_data_pad, tissue_id, seq_lengths, tissue_table, seq_table, W_pred, b_pred):
    raise NotImplementedError("write your pallas kernel here")


# speedup vs baseline: 71.2098x; 71.2098x over previous
import functools

import jax
import jax.numpy as jnp
from jax import lax
from jax.experimental import pallas as pl
from jax.experimental.pallas import tpu as pltpu
from jax.experimental.pallas import tpu_sc as plsc

_L = 16


def _rsqrt_nr(x):
    i = lax.bitcast_convert_type(x, jnp.int32)
    i = jnp.int32(0x5F3759DF) - lax.shift_right_logical(i, 1)
    y = lax.bitcast_convert_type(i, jnp.float32)
    hx = 0.5 * x
    for _ in range(4):
        y = y * (1.5 - hx * y * y)
    return y


def _scale(nrm2):
    nc = jnp.maximum(nrm2, 1e-12)
    n = nc * _rsqrt_nr(nc)
    return jnp.minimum(1.0, 2.0 / (n + 1e-7))


def _sc_body(S, RPW, DK, DT,
             rna_flat, tis_id, lens, tis_flat, seq_flat, w, b,
             out,
             len_v, tis_v, idx_v, tok_v, st_v, tt_v, w_v, b_v, y_v,
             sem):
    wid = lax.axis_index("s") * 2 + lax.axis_index("c")
    base = wid * RPW

    pltpu.sync_copy(lens.at[pl.ds(base, RPW)], len_v)
    pltpu.sync_copy(tis_id.at[pl.ds(base, RPW)], tis_v)
    pltpu.sync_copy(seq_flat, st_v)
    pltpu.sync_copy(tis_flat, tt_v)
    pltpu.sync_copy(w, w_v)
    pltpu.sync_copy(b, b_v)

    lane = jnp.arange(_L, dtype=jnp.int32)
    for g in range(RPW // _L):
        row = base + g * _L + lane
        flat = row * S + (len_v[pl.ds(g * _L, _L)] - 1)
        idx_v[pl.ds(g * _L, _L)] = flat

    pltpu.async_copy(rna_flat.at[idx_v], tok_v, sem).wait()

    bias = b_v[pl.ds(0, _L)]
    zero = jnp.zeros((_L,), jnp.float32)
    zi = jnp.zeros((_L,), jnp.int32)
    for g in range(RPW // _L):
        tok_g = tok_v[pl.ds(g * _L, _L)]
        tis_g = tis_v[pl.ds(g * _L, _L)]
        srow_base = tok_g * DK
        trow_base = tis_g * DT

        def sbody(d, carry):
            dot, nrm = carry
            v = plsc.load_gather(st_v, [srow_base + d])
            wd = plsc.load_gather(w_v, [zi + d])
            return dot + v * wd, nrm + v * v

        dot_s, nrm_s = lax.fori_loop(0, DK, sbody, (zero, zero))

        def tbody(d, carry):
            dot, nrm = carry
            v = plsc.load_gather(tt_v, [trow_base + d])
            wd = plsc.load_gather(w_v, [zi + DK + d])
            return dot + v * wd, nrm + v * v

        dot_t, nrm_t = lax.fori_loop(0, DT, tbody, (zero, zero))

        val = dot_s * _scale(nrm_s) + dot_t * _scale(nrm_t)
        y_v[pl.ds(g * _L, _L)] = jnp.where(tok_g != 0, val, 0.0) + bias

    pltpu.sync_copy(y_v, out.at[pl.ds(base, RPW)])


def kernel(rna_data_pad, tissue_id, seq_lengths, tissue_table, seq_table,
           W_pred, b_pred):
    B, S = rna_data_pad.shape
    VOCAB, DK = seq_table.shape
    NT, DT = tissue_table.shape
    NW = 32
    RPW = B // NW

    rna_flat = rna_data_pad.reshape(B * S)
    seq_flat = seq_table.reshape(VOCAB * DK)
    tis_flat = tissue_table.reshape(NT * DT)
    w = W_pred.reshape(-1)
    b_bcast = jnp.broadcast_to(b_pred, (_L,))

    mesh = plsc.VectorSubcoreMesh(core_axis_name="c", subcore_axis_name="s")
    run = pl.kernel(
        functools.partial(_sc_body, S, RPW, DK, DT),
        mesh=mesh,
        out_type=jax.ShapeDtypeStruct((B,), jnp.float32),
        compiler_params=pltpu.CompilerParams(needs_layout_passes=False),
        scratch_types=[
            pltpu.VMEM((RPW,), jnp.int32),
            pltpu.VMEM((RPW,), jnp.int32),
            pltpu.VMEM((RPW,), jnp.int32),
            pltpu.VMEM((RPW,), jnp.int32),
            pltpu.VMEM((VOCAB * DK,), jnp.float32),
            pltpu.VMEM((NT * DT,), jnp.float32),
            pltpu.VMEM((DK + DT,), jnp.float32),
            pltpu.VMEM((_L,), jnp.float32),
            pltpu.VMEM((RPW,), jnp.float32),
            pltpu.SemaphoreType.DMA,
        ],
    )
    y = run(rna_flat, tissue_id, seq_lengths, tis_flat, seq_flat, w, b_bcast)
    return y.reshape(B, 1)
